# Initial kernel scaffold; baseline (speedup 1.0000x reference)
import jax
import jax.numpy as jnp
from jax.experimental import pallas as pl


def kernel(gate_inputs, raw_inputs, keys, W_gate, W_expert):
    raise NotImplementedError("write your pallas kernel here")


# speedup vs baseline: 2.2806x; 2.2806x over previous
import jax
import jax.numpy as jnp
from jax.experimental import pallas as pl

T, XD, KD, E = 2048, 1024, 512, 8
TT = 1024


def _gate_body(gate_ref, keys_ref, wg_ref, scores_ref):
    q = jax.lax.dot_general(
        gate_ref[...], wg_ref[...], (((1,), (0,)), ((), ())),
        preferred_element_type=jnp.float32)
    logits = jax.lax.dot_general(
        q, keys_ref[...], (((1,), (1,)), ((), ())),
        preferred_element_type=jnp.float32)
    lane = jax.lax.broadcasted_iota(jnp.int32, logits.shape, 1)
    m1 = jnp.max(logits, axis=1, keepdims=True)
    idx1 = jnp.min(jnp.where(logits == m1, lane, E), axis=1, keepdims=True)
    rest = jnp.where(lane == idx1, -jnp.inf, logits)
    m2 = jnp.max(rest, axis=1, keepdims=True)
    idx2 = jnp.min(jnp.where(rest == m2, lane, E), axis=1, keepdims=True)
    ex = jnp.exp(m2 - m1)
    g1 = 1.0 / (1.0 + ex)
    g2 = ex * g1
    scores_ref[...] = (jnp.where(lane == idx1, g1, 0.0)
                       + jnp.where(lane == idx2, g2, 0.0))


def _expert_body(raw_ref, scores_ref, we_ref, out_ref):
    j = pl.program_id(1)
    lane = jax.lax.broadcasted_iota(jnp.int32, (TT, E), 1)
    col = jnp.sum(jnp.where(lane == j, scores_ref[...], 0.0),
                  axis=1, keepdims=True)
    contrib = col * jax.lax.dot_general(
        raw_ref[...], we_ref[0], (((1,), (0,)), ((), ())),
        preferred_element_type=jnp.float32)

    @pl.when(j == 0)
    def _init():
        out_ref[...] = contrib

    @pl.when(j > 0)
    def _acc():
        out_ref[...] += contrib


def kernel(gate_inputs, raw_inputs, keys, W_gate, W_expert):
    scores = pl.pallas_call(
        _gate_body,
        grid=(T // TT,),
        in_specs=[
            pl.BlockSpec((TT, XD), lambda i: (i, 0)),
            pl.BlockSpec((E, KD), lambda i: (0, 0)),
            pl.BlockSpec((XD, KD), lambda i: (0, 0)),
        ],
        out_specs=pl.BlockSpec((TT, E), lambda i: (i, 0)),
        out_shape=jax.ShapeDtypeStruct((T, E), jnp.float32),
    )(gate_inputs, keys, W_gate)

    out = pl.pallas_call(
        _expert_body,
        grid=(T // TT, E),
        in_specs=[
            pl.BlockSpec((TT, XD), lambda i, j: (i, 0)),
            pl.BlockSpec((TT, E), lambda i, j: (i, 0)),
            pl.BlockSpec((1, XD, XD), lambda i, j: (j, 0, 0)),
        ],
        out_specs=pl.BlockSpec((TT, XD), lambda i, j: (i, 0)),
        out_shape=jax.ShapeDtypeStruct((T, XD), jnp.float32),
    )(raw_inputs, scores, W_expert)
    return out, scores
